# Initial kernel scaffold; baseline (speedup 1.0000x reference)
import jax
import jax.numpy as jnp
from jax.experimental import pallas as pl


def kernel(x, node_logprob, batch, y, W1, b1, W2, b2):
    raise NotImplementedError("write your pallas kernel here")


# speedup vs baseline: 5.2378x; 5.2378x over previous
import jax
import jax.numpy as jnp
from jax import lax
from jax.experimental import pallas as pl
from jax.experimental.pallas import tpu as pltpu

_N, _D, _G, _DOUT = 10000, 512, 1024, 512
_BN = 1000
_NBLK = _N // _BN


def _body(x_ref, lp_ref, batch_ref, w1_ref, b1_ref, w2_ref, b2_ref,
          out_ref, acc_ref):
    p_all = jnp.exp(lp_ref[:, :])
    total = jnp.sum(p_all)

    acc_ref[:, :] = jnp.zeros((_G, _D), jnp.float32)

    def blk(j, _):
        xb = x_ref[pl.ds(j * _BN, _BN), :]
        pb = jnp.exp(lp_ref[pl.ds(j * _BN, _BN), :])
        bb = batch_ref[pl.ds(j * _BN, _BN), :]
        wb = xb * pb
        g_iota = lax.broadcasted_iota(jnp.int32, (_BN, _G), 1)
        s_t = (bb == g_iota).astype(jnp.float32)
        acc_ref[:, :] += lax.dot_general(
            s_t, wb, (((0,), (0,)), ((), ())),
            preferred_element_type=jnp.float32)
        return 0

    lax.fori_loop(0, _NBLK, blk, 0)

    pooled = acc_ref[:, :] / total
    h = jnp.maximum(
        jnp.dot(pooled, w1_ref[:, :], preferred_element_type=jnp.float32)
        + b1_ref[:, :], 0.0)
    out_ref[:, :] = jnp.dot(h, w2_ref[:, :],
                            preferred_element_type=jnp.float32) + b2_ref[:, :]


def kernel(x, node_logprob, batch, y, W1, b1, W2, b2):
    lp2 = node_logprob.reshape(_N, 1)
    b2d = batch.astype(jnp.int32).reshape(_N, 1)
    pred = pl.pallas_call(
        _body,
        out_shape=jax.ShapeDtypeStruct((_G, _DOUT), jnp.float32),
        scratch_shapes=[pltpu.VMEM((_G, _D), jnp.float32)],
    )(x, lp2, b2d, W1, b1.reshape(1, _D), W2, b2.reshape(1, _DOUT))
    return (pred, y)
